# baseline (speedup 1.0000x reference)
import functools

import jax
import jax.numpy as jnp
from jax import lax
from jax.experimental import pallas as pl
from jax.experimental.pallas import tpu as pltpu
from jax.experimental.pallas import tpu_sc as plsc

N_ROWS = 320000
N_COLS = 128
ADJ_K = 3
NC = 2
NS = 16
L = 16
NW = NC * NS
VR = 16
VPR = N_COLS // VR
ROWS_PER_TILE = N_ROWS // NW
RPB = 128
NBLK = ROWS_PER_TILE // RPB
TAIL = ROWS_PER_TILE - NBLK * RPB
BPC = 3
CHUNK = RPB * BPC
NCHUNKS = NBLK // BPC
GROUPS = CHUNK // L


def _sc_body(data_hbm, adj_hbm, out_idx_hbm, out_hbm, buf0, buf1, buf2,
             tail_v, idx_v, acc_v, cols_v, sem0, sem1, sem2, sem_t):
    wid = lax.axis_index("s") * NC + lax.axis_index("c")
    base = wid * ROWS_PER_TILE
    pltpu.sync_copy(adj_hbm, cols_v.at[0, pl.ds(0, ADJ_K)])
    pltpu.sync_copy(out_idx_hbm, cols_v.at[1, pl.ds(0, 1)])
    bufs = (buf0, buf1, buf2)
    sems = (sem0, sem1, sem2)

    zero = jnp.zeros((L,), jnp.float32)
    one = jnp.ones((L,), jnp.float32)
    iota = lax.iota(jnp.int32, L)

    av = cols_v[0, :]
    ov = cols_v[1, :]
    csc = [av[0], av[1], av[2], ov[0]]
    cgran = [lax.shift_right_logical(c, 4) for c in csc]
    clane = [jnp.broadcast_to(jnp.bitwise_and(c, 15), (L,)) for c in csc]

    for j in range(4):
        vtail = (base + NBLK * RPB + iota) * VPR + cgran[j]
        pltpu.async_copy(data_hbm.at[vtail], tail_v.at[j], sem_t)

    jfull = [jnp.full((L,), j, jnp.int32) for j in range(4)]

    def build_block(b, _):
        for g8 in range(RPB // L):
            rows = b * RPB + g8 * L + iota
            vrow = (base + rows) * VPR
            for j in range(4):
                plsc.store_scatter(idx_v, [jfull[j], rows], vrow + cgran[j])
        return 0

    lax.fori_loop(0, BPC, build_block, 0)

    def _dmas(ci, bi):
        out = []
        for j in range(4):
            for b in range(BPC):
                blk0 = (ci * BPC + b) * RPB
                src = data_hbm.at[idx_v.at[j, pl.ds(blk0, RPB)]]
                dst = bufs[bi].at[j, pl.ds(b * RPB, RPB)]
                out.append((src, dst))
        return out

    def _start_chunk(ci, bi):
        for src, dst in _dmas(ci, bi):
            pltpu.async_copy(src, dst, sems[bi])

    def _wait_chunk(ci, bi):
        for src, dst in _dmas(ci, bi):
            pltpu.make_async_copy(src, dst, sems[bi]).wait()

    def _accumulate(bufy, bufa, g, acc):
        rows = g * L + iota
        a0 = plsc.load_gather(bufa[0], [rows, clane[0]])
        a1 = plsc.load_gather(bufa[1], [rows, clane[1]])
        a2 = plsc.load_gather(bufa[2], [rows, clane[2]])
        y = plsc.load_gather(bufy, [rows, clane[3]])
        sid = a0 + 2.0 * a1 + 4.0 * a2
        acc = list(acc)
        for s in range(8):
            m = sid == float(s)
            acc[s] = acc[s] + jnp.where(m, y, zero)
            acc[s + 8] = acc[s + 8] + jnp.where(m, one, zero)
        return tuple(acc)

    def _process(bi, acc):
        def group_body(g, a, _buf=bufs[bi]):
            return _accumulate(_buf.at[3], [_buf.at[0], _buf.at[1],
                                            _buf.at[2]], g, a)
        return lax.fori_loop(0, GROUPS, group_body, acc, unroll=4)

    _start_chunk(0, 0)
    lax.fori_loop(BPC, 2 * BPC, build_block, 0)
    _start_chunk(1, 1)
    lax.fori_loop(2 * BPC, NBLK, build_block, 0)

    def chunk_triple(ct, acc):
        for b in range(3):
            ci = ct * 3 + b
            nb = (b + 2) % 3

            @pl.when(ci + 2 < NCHUNKS)
            def _():
                _start_chunk(ci + 2, nb)

            _wait_chunk(ci, b)
            acc = _process(b, acc)
        return acc

    acc0 = tuple(zero for _ in range(16))
    acc = lax.fori_loop(0, NCHUNKS // 3, chunk_triple, acc0)
    for ci in range((NCHUNKS // 3) * 3, NCHUNKS):
        _wait_chunk(ci, ci % 3)
        acc = _process(ci % 3, acc)

    for j in range(4):
        vtail = (base + NBLK * RPB + iota) * VPR + cgran[j]
        pltpu.make_async_copy(data_hbm.at[vtail], tail_v.at[j], sem_t).wait()
    acc = _accumulate(tail_v.at[3], [tail_v.at[0], tail_v.at[1],
                                     tail_v.at[2]], 0, acc)

    for s in range(16):
        acc_v[s, :] = acc[s]
    pltpu.sync_copy(acc_v, out_hbm.at[wid])


def _make_sc_call(interpret=False):
    return pl.kernel(
        _sc_body,
        out_type=jax.ShapeDtypeStruct((NW, 16, L), jnp.float32),
        mesh=plsc.VectorSubcoreMesh(
            core_axis_name="c", subcore_axis_name="s",
            num_cores=NC, num_subcores=NS),
        scratch_types=[
            pltpu.VMEM((4, CHUNK, VR), jnp.float32),
            pltpu.VMEM((4, CHUNK, VR), jnp.float32),
            pltpu.VMEM((4, CHUNK, VR), jnp.float32),
            pltpu.VMEM((4, TAIL, VR), jnp.float32),
            pltpu.VMEM((4, NBLK * RPB), jnp.int32),
            pltpu.VMEM((16, L), jnp.float32),
            pltpu.VMEM((2, L), jnp.int32),
            pltpu.SemaphoreType.DMA,
            pltpu.SemaphoreType.DMA,
            pltpu.SemaphoreType.DMA,
            pltpu.SemaphoreType.DMA,
        ],
        compiler_params=pltpu.CompilerParams(
            needs_layout_passes=False, use_tc_tiling_on_sc=False),
        interpret=interpret,
    )


def _combine_body(p_ref, o_ref):
    acc = p_ref[0]
    for i in range(1, NW):
        acc = acc + p_ref[i]
    t = jnp.sum(acc, axis=1, keepdims=True)
    sums = t[0:8]
    counts = t[8:16]
    means = sums / jnp.maximum(counts, 1.0)
    effects = jnp.where(counts > 0, means * counts / float(N_ROWS), 0.0)
    o_ref[0, 0] = jnp.sum(effects)


_combine = pl.pallas_call(
    _combine_body,
    out_shape=jax.ShapeDtypeStruct((1, 1), jnp.float32),
    in_specs=[pl.BlockSpec(memory_space=pltpu.VMEM)],
    out_specs=pl.BlockSpec(memory_space=pltpu.SMEM),
)


def kernel(data, treatment_idx, outcome_idx, adjustment_set):
    adj = adjustment_set.astype(jnp.int32).reshape(ADJ_K)
    oidx = jnp.asarray(outcome_idx, jnp.int32).reshape(1)
    data16 = data.reshape(N_ROWS * VPR, VR)
    partials = _make_sc_call()(data16, adj, oidx)
    return _combine(partials)[0, 0]


# speedup vs baseline: 1.0204x; 1.0204x over previous
import functools

import jax
import jax.numpy as jnp
from jax import lax
from jax.experimental import pallas as pl
from jax.experimental.pallas import tpu as pltpu
from jax.experimental.pallas import tpu_sc as plsc

N_ROWS = 320000
N_COLS = 128
ADJ_K = 3
NC = 2
NS = 16
L = 16
NW = NC * NS
VR = 16
VPR = N_COLS // VR
ROWS_PER_TILE = N_ROWS // NW
RPB = 128
NBLK = ROWS_PER_TILE // RPB
TAIL = ROWS_PER_TILE - NBLK * RPB
BPC = 3
CHUNK = RPB * BPC
NCHUNKS = NBLK // BPC
GROUPS = CHUNK // L


def _sc_body(data_hbm, adj_hbm, out_idx_hbm, out_hbm, buf0, buf1, buf2,
             tail_v, idx_v, acc_v, cols_v, sem0, sem1, sem2, sem_t):
    wid = lax.axis_index("s") * NC + lax.axis_index("c")
    base = wid * ROWS_PER_TILE
    pltpu.sync_copy(adj_hbm, cols_v.at[0, pl.ds(0, ADJ_K)])
    pltpu.sync_copy(out_idx_hbm, cols_v.at[1, pl.ds(0, 1)])
    bufs = (buf0, buf1, buf2)
    sems = (sem0, sem1, sem2)

    zero = jnp.zeros((L,), jnp.float32)
    one = jnp.ones((L,), jnp.float32)
    iota = lax.iota(jnp.int32, L)

    av = cols_v[0, :]
    ov = cols_v[1, :]
    csc = [av[0], av[1], av[2], ov[0]]
    cgran = [lax.shift_right_logical(c, 4) for c in csc]
    clane = [jnp.broadcast_to(jnp.bitwise_and(c, 15), (L,)) for c in csc]

    for j in range(4):
        vtail = (base + NBLK * RPB + iota) * VPR + cgran[j]
        pltpu.async_copy(data_hbm.at[vtail], tail_v.at[j], sem_t)

    jfull = [jnp.full((L,), j, jnp.int32) for j in range(4)]

    def build_block(b, _):
        for g8 in range(RPB // L):
            rows = b * RPB + g8 * L + iota
            vrow = (base + rows) * VPR
            for j in range(4):
                plsc.store_scatter(idx_v, [jfull[j], rows], vrow + cgran[j])
        return 0

    lax.fori_loop(0, BPC, build_block, 0)

    def _dmas(ci, bi):
        out = []
        for j in range(4):
            for b in range(BPC):
                blk0 = (ci * BPC + b) * RPB
                src = data_hbm.at[idx_v.at[j, pl.ds(blk0, RPB)]]
                dst = bufs[bi].at[j, pl.ds(b * RPB, RPB)]
                out.append((src, dst))
        return out

    def _start_chunk(ci, bi):
        for src, dst in _dmas(ci, bi):
            pltpu.async_copy(src, dst, sems[bi])

    def _wait_chunk(ci, bi):
        for src, dst in _dmas(ci, bi):
            pltpu.make_async_copy(src, dst, sems[bi]).wait()

    def _accumulate(bufy, bufa, g, acc):
        rows = g * L + iota
        a0 = plsc.load_gather(bufa[0], [rows, clane[0]])
        a1 = plsc.load_gather(bufa[1], [rows, clane[1]])
        a2 = plsc.load_gather(bufa[2], [rows, clane[2]])
        y = plsc.load_gather(bufy, [rows, clane[3]])
        sid = a0 + 2.0 * a1 + 4.0 * a2
        acc = list(acc)
        for s in range(8):
            m = sid == float(s)
            acc[s] = acc[s] + jnp.where(m, y, zero)
            acc[s + 8] = acc[s + 8] + jnp.where(m, one, zero)
        return tuple(acc)

    def _process(bi, acc):
        def group_body(g, a, _buf=bufs[bi]):
            return _accumulate(_buf.at[3], [_buf.at[0], _buf.at[1],
                                            _buf.at[2]], g, a)
        return lax.fori_loop(0, GROUPS, group_body, acc)

    _start_chunk(0, 0)
    lax.fori_loop(BPC, 2 * BPC, build_block, 0)
    _start_chunk(1, 1)
    lax.fori_loop(2 * BPC, NBLK, build_block, 0)

    def chunk_triple(ct, acc):
        for b in range(3):
            ci = ct * 3 + b
            nb = (b + 2) % 3

            @pl.when(ci + 2 < NCHUNKS)
            def _():
                _start_chunk(ci + 2, nb)

            _wait_chunk(ci, b)
            acc = _process(b, acc)
        return acc

    acc0 = tuple(zero for _ in range(16))
    acc = lax.fori_loop(0, NCHUNKS // 3, chunk_triple, acc0)
    for ci in range((NCHUNKS // 3) * 3, NCHUNKS):
        _wait_chunk(ci, ci % 3)
        acc = _process(ci % 3, acc)

    for j in range(4):
        vtail = (base + NBLK * RPB + iota) * VPR + cgran[j]
        pltpu.make_async_copy(data_hbm.at[vtail], tail_v.at[j], sem_t).wait()
    acc = _accumulate(tail_v.at[3], [tail_v.at[0], tail_v.at[1],
                                     tail_v.at[2]], 0, acc)

    for s in range(16):
        acc_v[s, :] = acc[s]
    pltpu.sync_copy(acc_v, out_hbm.at[wid])


def _make_sc_call(interpret=False):
    return pl.kernel(
        _sc_body,
        out_type=jax.ShapeDtypeStruct((NW, 16, L), jnp.float32),
        mesh=plsc.VectorSubcoreMesh(
            core_axis_name="c", subcore_axis_name="s",
            num_cores=NC, num_subcores=NS),
        scratch_types=[
            pltpu.VMEM((4, CHUNK, VR), jnp.float32),
            pltpu.VMEM((4, CHUNK, VR), jnp.float32),
            pltpu.VMEM((4, CHUNK, VR), jnp.float32),
            pltpu.VMEM((4, TAIL, VR), jnp.float32),
            pltpu.VMEM((4, NBLK * RPB), jnp.int32),
            pltpu.VMEM((16, L), jnp.float32),
            pltpu.VMEM((2, L), jnp.int32),
            pltpu.SemaphoreType.DMA,
            pltpu.SemaphoreType.DMA,
            pltpu.SemaphoreType.DMA,
            pltpu.SemaphoreType.DMA,
        ],
        compiler_params=pltpu.CompilerParams(
            needs_layout_passes=False, use_tc_tiling_on_sc=False),
        interpret=interpret,
    )


def _combine_body(p_ref, o_ref):
    acc = p_ref[0]
    for i in range(1, NW):
        acc = acc + p_ref[i]
    t = jnp.sum(acc, axis=1, keepdims=True)
    sums = t[0:8]
    counts = t[8:16]
    means = sums / jnp.maximum(counts, 1.0)
    effects = jnp.where(counts > 0, means * counts / float(N_ROWS), 0.0)
    o_ref[0, 0] = jnp.sum(effects)


_combine = pl.pallas_call(
    _combine_body,
    out_shape=jax.ShapeDtypeStruct((1, 1), jnp.float32),
    in_specs=[pl.BlockSpec(memory_space=pltpu.VMEM)],
    out_specs=pl.BlockSpec(memory_space=pltpu.SMEM),
)


def kernel(data, treatment_idx, outcome_idx, adjustment_set):
    adj = adjustment_set.astype(jnp.int32).reshape(ADJ_K)
    oidx = jnp.asarray(outcome_idx, jnp.int32).reshape(1)
    data16 = data.reshape(N_ROWS * VPR, VR)
    partials = _make_sc_call()(data16, adj, oidx)
    return _combine(partials)[0, 0]


# speedup vs baseline: 1.0211x; 1.0007x over previous
import jax
import jax.numpy as jnp
from jax import lax
from jax.experimental import pallas as pl
from jax.experimental.pallas import tpu as pltpu
from jax.experimental.pallas import tpu_sc as plsc

N_ROWS = 320000
N_COLS = 128
ADJ_K = 3
NC = 2
NS = 16
L = 16
NW = NC * NS
VR = 16
VPR = N_COLS // VR
ROWS_PER_TILE = N_ROWS // NW
RPB = 128
NBLK = ROWS_PER_TILE // RPB
TAIL = ROWS_PER_TILE - NBLK * RPB
BPC = 3
CHUNK = RPB * BPC
NCHUNKS = NBLK // BPC
GROUPS = CHUNK // L


def _sc_body(data_hbm, adj_hbm, out_idx_hbm, out_hbm, buf0, buf1, buf2,
             tail_v, idx_v, acc_v, cols_v, sem0, sem1, sem2, sem_t):
    wid = lax.axis_index("s") * NC + lax.axis_index("c")
    base = wid * ROWS_PER_TILE
    pltpu.sync_copy(adj_hbm, cols_v.at[0, pl.ds(0, ADJ_K)])
    pltpu.sync_copy(out_idx_hbm, cols_v.at[1, pl.ds(0, 1)])
    bufs = (buf0, buf1, buf2)
    sems = (sem0, sem1, sem2)

    zero = jnp.zeros((L,), jnp.float32)
    one = jnp.ones((L,), jnp.float32)
    iota = lax.iota(jnp.int32, L)

    av = cols_v[0, :]
    ov = cols_v[1, :]
    csc = [av[0], av[1], av[2], ov[0]]
    cgran = [lax.shift_right_logical(c, 4) for c in csc]
    clane = [jnp.broadcast_to(jnp.bitwise_and(c, 15), (L,)) for c in csc]

    for j in range(4):
        vtail = (base + NBLK * RPB + iota) * VPR + cgran[j]
        pltpu.async_copy(data_hbm.at[vtail], tail_v.at[j], sem_t)

    jfull = [jnp.full((L,), j, jnp.int32) for j in range(4)]

    def build_block(b, _):
        for g8 in range(RPB // L):
            rows = b * RPB + g8 * L + iota
            vrow = (base + rows) * VPR
            for j in range(4):
                plsc.store_scatter(idx_v, [jfull[j], rows], vrow + cgran[j])
        return 0

    lax.fori_loop(0, BPC, build_block, 0)

    def _dmas(ci, bi):
        out = []
        for j in range(4):
            for b in range(BPC):
                blk0 = (ci * BPC + b) * RPB
                src = data_hbm.at[idx_v.at[j, pl.ds(blk0, RPB)]]
                dst = bufs[bi].at[j, pl.ds(b * RPB, RPB)]
                out.append((src, dst))
        return out

    def _start_chunk(ci, bi):
        for src, dst in _dmas(ci, bi):
            pltpu.async_copy(src, dst, sems[bi])

    def _wait_chunk(ci, bi):
        for src, dst in _dmas(ci, bi):
            pltpu.make_async_copy(src, dst, sems[bi]).wait()

    def _accumulate(bufy, bufa, g, acc):
        rows = g * L + iota
        a0 = plsc.load_gather(bufa[0], [rows, clane[0]])
        a1 = plsc.load_gather(bufa[1], [rows, clane[1]])
        a2 = plsc.load_gather(bufa[2], [rows, clane[2]])
        y = plsc.load_gather(bufy, [rows, clane[3]])
        sid = a0 + 2.0 * a1 + 4.0 * a2
        acc = list(acc)
        for s in range(8):
            m = sid == float(s)
            acc[s] = acc[s] + jnp.where(m, y, zero)
            acc[s + 8] = acc[s + 8] + jnp.where(m, one, zero)
        return tuple(acc)

    def _process(bi, acc):
        def group_body(g, a, _buf=bufs[bi]):
            return _accumulate(_buf.at[3], [_buf.at[0], _buf.at[1],
                                            _buf.at[2]], g, a)
        return lax.fori_loop(0, GROUPS, group_body, acc)

    _start_chunk(0, 0)
    lax.fori_loop(BPC, 2 * BPC, build_block, 0)
    _start_chunk(1, 1)
    lax.fori_loop(2 * BPC, NBLK, build_block, 0)

    def chunk_triple(ct, acc):
        for b in range(3):
            ci = ct * 3 + b
            nb = (b + 2) % 3

            @pl.when(ci + 2 < NCHUNKS)
            def _():
                _start_chunk(ci + 2, nb)

            _wait_chunk(ci, b)
            acc = _process(b, acc)
        return acc

    acc0 = tuple(zero for _ in range(16))
    acc = lax.fori_loop(0, NCHUNKS // 3, chunk_triple, acc0)
    for ci in range((NCHUNKS // 3) * 3, NCHUNKS):
        _wait_chunk(ci, ci % 3)
        acc = _process(ci % 3, acc)

    for j in range(4):
        vtail = (base + NBLK * RPB + iota) * VPR + cgran[j]
        pltpu.make_async_copy(data_hbm.at[vtail], tail_v.at[j], sem_t).wait()
    acc = _accumulate(tail_v.at[3], [tail_v.at[0], tail_v.at[1],
                                     tail_v.at[2]], 0, acc)

    for s in range(16):
        acc_v[s, :] = acc[s]
    pltpu.sync_copy(acc_v, out_hbm.at[wid])


def _make_sc_call(interpret=False):
    return pl.kernel(
        _sc_body,
        out_type=jax.ShapeDtypeStruct((NW, 16, L), jnp.float32),
        mesh=plsc.VectorSubcoreMesh(
            core_axis_name="c", subcore_axis_name="s",
            num_cores=NC, num_subcores=NS),
        scratch_types=[
            pltpu.VMEM((4, CHUNK, VR), jnp.float32),
            pltpu.VMEM((4, CHUNK, VR), jnp.float32),
            pltpu.VMEM((4, CHUNK, VR), jnp.float32),
            pltpu.VMEM((4, TAIL, VR), jnp.float32),
            pltpu.VMEM((4, NBLK * RPB), jnp.int32),
            pltpu.VMEM((16, L), jnp.float32),
            pltpu.VMEM((2, L), jnp.int32),
            pltpu.SemaphoreType.DMA,
            pltpu.SemaphoreType.DMA,
            pltpu.SemaphoreType.DMA,
            pltpu.SemaphoreType.DMA,
        ],
        compiler_params=pltpu.CompilerParams(
            needs_layout_passes=False, use_tc_tiling_on_sc=False),
        interpret=interpret,
    )


def _combine_body(p_ref, o_ref):
    acc = p_ref[0]
    for i in range(1, NW):
        acc = acc + p_ref[i]
    t = jnp.sum(acc, axis=1, keepdims=True)
    sums = t[0:8]
    counts = t[8:16]
    means = sums / jnp.maximum(counts, 1.0)
    effects = jnp.where(counts > 0, means * counts / float(N_ROWS), 0.0)
    o_ref[0, 0] = jnp.sum(effects)


_combine = pl.pallas_call(
    _combine_body,
    out_shape=jax.ShapeDtypeStruct((1, 1), jnp.float32),
    in_specs=[pl.BlockSpec(memory_space=pltpu.VMEM)],
    out_specs=pl.BlockSpec(memory_space=pltpu.SMEM),
)


def kernel(data, treatment_idx, outcome_idx, adjustment_set):
    adj = adjustment_set.astype(jnp.int32).reshape(ADJ_K)
    oidx = jnp.asarray(outcome_idx, jnp.int32).reshape(1)
    data16 = data.reshape(N_ROWS * VPR, VR)
    partials = _make_sc_call()(data16, adj, oidx)
    return _combine(partials)[0, 0]
